# Initial kernel scaffold; baseline (speedup 1.0000x reference)
import jax
import jax.numpy as jnp
from jax.experimental import pallas as pl


def kernel(sources, destinations, node_features, start_idx, end_idx, timestamps, s_h_nodes, t_h_nodes, s_h_times, t_h_times, s_his_mask, t_his_mask, neg_src_idx, neg_dst_idx, delta_s, delta_t, W, a, g_w, g_b, mlp_w1, mlp_b1, mlp_w2, mlp_b2):
    raise NotImplementedError("write your pallas kernel here")


# speedup vs baseline: 1.2623x; 1.2623x over previous
import functools

import jax
import jax.numpy as jnp
from jax import lax
from jax.experimental import pallas as pl
from jax.experimental.pallas import tpu as pltpu
from jax.experimental.pallas import tpu_sc as plsc

NODE_DIM = 100000
FEAT_DIM = 128
EMB = 128
BATCH = 4096
HIST = 20
NEG = 5
INDEX = 200000
MAX_T = 100000.0
SIGMA = 1.0

NW = 32
CHUNK = 128
NBUF = 6

ROWS_W = BATCH * (4 + NEG * 2 + HIST * 2)
ROWS_E = BATCH * 2 + BATCH * NEG * 2


def _mlp_body(x_ref, w1_ref, b1_ref, w2_ref, b2_ref, w_ref, emb_ref, embw_ref):
    x = x_ref[...]
    h = jnp.maximum(
        jnp.dot(x, w1_ref[...], preferred_element_type=jnp.float32) + b1_ref[...],
        0.0)
    e = jnp.dot(h, w2_ref[...], preferred_element_type=jnp.float32) + b2_ref[...]
    emb_ref[...] = e
    embw_ref[...] = jnp.dot(e, w_ref[...], preferred_element_type=jnp.float32)


def _run_mlp(node_features, w1p, b1p, w2p, b2p, W):
    R = 1000
    G = NODE_DIM // R
    return pl.pallas_call(
        _mlp_body,
        grid=(G,),
        in_specs=[
            pl.BlockSpec((R, FEAT_DIM), lambda i: (i, 0)),
            pl.BlockSpec((FEAT_DIM, 128), lambda i: (0, 0)),
            pl.BlockSpec((1, 128), lambda i: (0, 0)),
            pl.BlockSpec((128, EMB), lambda i: (0, 0)),
            pl.BlockSpec((1, EMB), lambda i: (0, 0)),
            pl.BlockSpec((EMB, EMB), lambda i: (0, 0)),
        ],
        out_specs=[
            pl.BlockSpec((R, EMB), lambda i: (i, 0)),
            pl.BlockSpec((R, EMB), lambda i: (i, 0)),
        ],
        out_shape=[
            jax.ShapeDtypeStruct((NODE_DIM, EMB), jnp.float32),
            jax.ShapeDtypeStruct((NODE_DIM, EMB), jnp.float32),
        ],
    )(node_features, w1p, b1p, w2p, b2p, W)


def _sc_gather(embW, emb, idxw3, idxe3, src2, dst2, delta_s, delta_t):
    kw = idxw3.shape[1]
    ke = idxe3.shape[1]
    rows_w = NW * kw * CHUNK
    rows_e = NW * ke * CHUNK
    mesh = plsc.VectorSubcoreMesh(core_axis_name="c", subcore_axis_name="s")

    @functools.partial(
        pl.kernel,
        mesh=mesh,
        out_type=[
            jax.ShapeDtypeStruct((rows_w, EMB), jnp.float32),
            jax.ShapeDtypeStruct((rows_e, EMB), jnp.float32),
            jax.ShapeDtypeStruct((BATCH,), jnp.float32),
            jax.ShapeDtypeStruct((BATCH,), jnp.float32),
        ],
        scratch_types=[
            pltpu.VMEM((kw, CHUNK), jnp.int32),
            pltpu.VMEM((ke, CHUNK), jnp.int32),
            pltpu.VMEM((NBUF, CHUNK, EMB), jnp.float32),
            pltpu.VMEM((CHUNK,), jnp.int32),
            pltpu.VMEM((CHUNK,), jnp.float32),
            pltpu.SemaphoreType.DMA((NBUF,)),
            pltpu.SemaphoreType.DMA((NBUF,)),
            pltpu.SemaphoreType.DMA,
        ],
    )
    def k(embw_hbm, emb_hbm, idxw_hbm, idxe_hbm, src_hbm, dst_hbm,
          ds_hbm, dt_hbm, gw_out, ge_out, dsg_out, dtg_out,
          idxw_v, idxe_v, bufs, didx, dval, gsems, osems, dsem):
        wid = lax.axis_index("s") * 2 + lax.axis_index("c")

        def gather_set(table_hbm, idx_v, out_hbm, n_chunks, base_rows):
            handles_g = [None] * NBUF
            handles_o = [None] * NBUF
            depth = min(NBUF, n_chunks)
            for j in range(depth):
                handles_g[j] = pltpu.async_copy(
                    table_hbm.at[idx_v.at[j]], bufs.at[j], gsems.at[j])
            for j in range(n_chunks):
                slot = j % NBUF
                handles_g[slot].wait()
                handles_o[slot] = pltpu.async_copy(
                    bufs.at[slot],
                    out_hbm.at[pl.ds(base_rows + j * CHUNK, CHUNK)],
                    osems.at[slot])
                nx = j + NBUF
                if nx < n_chunks:
                    handles_o[slot].wait()
                    handles_g[slot] = pltpu.async_copy(
                        table_hbm.at[idx_v.at[nx]], bufs.at[slot],
                        gsems.at[slot])
            for j in range(max(0, n_chunks - NBUF), n_chunks):
                slot = j % NBUF
                if handles_o[slot] is not None:
                    handles_o[slot].wait()
                    handles_o[slot] = None

        pltpu.sync_copy(idxw_hbm.at[wid], idxw_v)
        pltpu.sync_copy(idxe_hbm.at[wid], idxe_v)
        gather_set(embw_hbm, idxw_v, gw_out, kw, wid * kw * CHUNK)
        gather_set(emb_hbm, idxe_v, ge_out, ke, wid * ke * CHUNK)

        pltpu.sync_copy(src_hbm.at[wid], didx)
        pltpu.async_copy(ds_hbm.at[didx], dval, dsem).wait()
        pltpu.sync_copy(dval, dsg_out.at[pl.ds(wid * CHUNK, CHUNK)])
        pltpu.sync_copy(dst_hbm.at[wid], didx)
        pltpu.async_copy(dt_hbm.at[didx], dval, dsem).wait()
        pltpu.sync_copy(dval, dtg_out.at[pl.ds(wid * CHUNK, CHUNK)])

    return k(embW, emb, idxw3, idxe3, src2, dst2, delta_s, delta_t)


def _att_body(p0_ref, p1_ref, sw_ref, tw_ref, n0_ref, n1_ref, n0w_ref, n1w_ref,
              hws_ref, hwt_ref, et_ref, sht_ref, tht_ref, sm_ref, tm_ref,
              ds_ref, dt_ref, a2_ref, gw_ref, gb_ref, ew_ref, loss_ref):
    p0 = p0_ref[...]
    p1 = p1_ref[...]
    sw = sw_ref[...]
    tw = tw_ref[...]
    a_top = a2_ref[0:1, :].reshape(1, 1, EMB)
    a_bot = a2_ref[1:2, :].reshape(1, 1, EMB)
    gwv = gw_ref[...].reshape(1, 1, EMB)
    gbv = gb_ref[...].reshape(1, 1, 1)
    ds = ds_ref[...]
    dt_ = dt_ref[...]

    etn = et_ref[...] / MAX_T
    shn = sht_ref[...] / MAX_T
    thn = tht_ref[...] / MAX_T
    dts = jnp.abs(etn - shn)
    dtt = jnp.abs(etn - thn)

    hws = hws_ref[...]
    hwt = hwt_ref[...]
    swa = jnp.sum(sw * a_top, axis=2, keepdims=True)
    twa = jnp.sum(tw * a_top, axis=2, keepdims=True)
    hsa = jnp.sum(hws * a_bot, axis=2, keepdims=True)
    hta = jnp.sum(hwt * a_bot, axis=2, keepdims=True)

    def lrelu(x):
        return jnp.where(x >= 0, x, 0.2 * x)

    sims = lrelu(jnp.exp(-ds * dts) * (swa + hsa))
    simt = lrelu(jnp.exp(-ds * dtt) * (twa + hta))

    ms = jnp.max(sims, axis=1, keepdims=True)
    es = jnp.exp(sims - ms)
    atts = es / jnp.sum(es, axis=1, keepdims=True)
    mt = jnp.max(simt, axis=1, keepdims=True)
    et2 = jnp.exp(simt - mt)
    attt = et2 / jnp.sum(et2, axis=1, keepdims=True)

    shat_i = jnp.sum((atts * sm_ref[...]) * hws, axis=1, keepdims=True)
    that_i = jnp.sum((attt * tm_ref[...]) * hwt, axis=1, keepdims=True)

    mdts = jnp.mean(dts, axis=1, keepdims=True)
    mdtt = jnp.mean(dtt, axis=1, keepdims=True)
    gs = shat_i * jnp.exp(-ds * mdts)
    gt = that_i * jnp.exp(-dt_ * mdtt)
    gas = jnp.tanh(jnp.sum(gs * gwv, axis=2, keepdims=True) + gbv)
    gat = jnp.tanh(jnp.sum(gt * gwv, axis=2, keepdims=True) + gbv)
    mg = jnp.maximum(gas, gat)
    egs = jnp.exp(gas - mg)
    egt = jnp.exp(gat - mg)
    den = egs + egt
    shat = sw + (egs / den) * shat_i
    that = tw + (egt / den) * that_i

    pos = jnp.maximum(jnp.sum(p0 * p1, axis=2, keepdims=True), 0.0)
    negs = jnp.maximum(jnp.sum(p0 * n1_ref[...], axis=2, keepdims=True), 0.0)
    negt = jnp.maximum(jnp.sum(p1 * n0_ref[...], axis=2, keepdims=True), 0.0)

    psim = jnp.sum(shat * that, axis=2, keepdims=True)
    snsim = jnp.sum(shat * n1w_ref[...], axis=2, keepdims=True)
    tnsim = jnp.sum(that * n0w_ref[...], axis=2, keepdims=True)

    def sig(x):
        return 1.0 / (1.0 + jnp.exp(-x))

    pls = -jnp.log(sig(psim / SIGMA) + 1e-6) * (pos - 1.0) ** 2
    snl = -jnp.log(sig(-snsim / SIGMA) + 1e-6) * negs ** 2
    tnl = -jnp.log(sig(-tnsim / SIGMA) + 1e-6) * negt ** 2
    partial = (jnp.sum(pls) / INDEX
               + (jnp.sum(snl) + jnp.sum(tnl)) / (INDEX * NEG))

    i = pl.program_id(0)

    @pl.when(i == 0)
    def _():
        loss_ref[...] = jnp.zeros_like(loss_ref)

    loss_ref[...] += jnp.reshape(partial, (1, 1))
    ew_ref[...] = pos


def _run_att(p0, p1, sw, tw, n0, n1, n0w, n1w, hws, hwt,
             et, sht, tht, sm, tm, dsg, dtg, a2, gwT, gb2):
    TB = 128
    G = BATCH // TB
    b3 = lambda shp: pl.BlockSpec((TB,) + shp, lambda i: (i, 0, 0))
    full2 = lambda shp: pl.BlockSpec(shp, lambda i: (0, 0))
    return pl.pallas_call(
        _att_body,
        grid=(G,),
        in_specs=[
            b3((1, EMB)), b3((1, EMB)), b3((1, EMB)), b3((1, EMB)),
            b3((NEG, EMB)), b3((NEG, EMB)), b3((NEG, EMB)), b3((NEG, EMB)),
            b3((HIST, EMB)), b3((HIST, EMB)),
            b3((1, 1)), b3((HIST, 1)), b3((HIST, 1)),
            b3((HIST, 1)), b3((HIST, 1)),
            b3((1, 1)), b3((1, 1)),
            full2((2, EMB)), full2((1, EMB)), full2((1, 1)),
        ],
        out_specs=[
            b3((1, 1)),
            pl.BlockSpec((1, 1), lambda i: (0, 0)),
        ],
        out_shape=[
            jax.ShapeDtypeStruct((BATCH, 1, 1), jnp.float32),
            jax.ShapeDtypeStruct((1, 1), jnp.float32),
        ],
    )(p0, p1, sw, tw, n0, n1, n0w, n1w, hws, hwt,
      et, sht, tht, sm, tm, dsg, dtg, a2, gwT, gb2)


def kernel(sources, destinations, node_features, start_idx, end_idx,
           timestamps, s_h_nodes, t_h_nodes, s_h_times, t_h_times,
           s_his_mask, t_his_mask, neg_src_idx, neg_dst_idx,
           delta_s, delta_t, W, a, g_w, g_b,
           mlp_w1, mlp_b1, mlp_w2, mlp_b2):
    B = sources.shape[0]

    w1p = jnp.zeros((FEAT_DIM, 128), jnp.float32).at[:, :mlp_w1.shape[1]].set(mlp_w1)
    b1p = jnp.zeros((1, 128), jnp.float32).at[0, :mlp_b1.shape[0]].set(mlp_b1)
    w2p = jnp.zeros((128, EMB), jnp.float32).at[:mlp_w2.shape[0], :].set(mlp_w2)
    b2p = mlp_b2.reshape(1, EMB)

    node_emb, embW = _run_mlp(node_features, w1p, b1p, w2p, b2p, W)

    begin = end_idx - B
    sl = lambda x: lax.dynamic_slice_in_dim(x, begin, B, axis=0)
    e_t = sl(timestamps).reshape(B, 1, 1)
    sh_n = sl(s_h_nodes)
    th_n = sl(t_h_nodes)
    sh_t = sl(s_h_times).reshape(B, HIST, 1)
    th_t = sl(t_h_times).reshape(B, HIST, 1)
    s_m = sl(s_his_mask).reshape(B, HIST, 1)
    t_m = sl(t_his_mask).reshape(B, HIST, 1)

    idx_w = jnp.concatenate([
        sources, destinations, neg_src_idx, neg_dst_idx,
        sh_n.reshape(-1), th_n.reshape(-1),
    ]).astype(jnp.int32)
    idx_e = jnp.concatenate([
        sources, destinations, neg_src_idx, neg_dst_idx,
    ]).astype(jnp.int32)
    kw = idx_w.shape[0] // (NW * CHUNK)
    ke = idx_e.shape[0] // (NW * CHUNK)
    idxw3 = idx_w.reshape(NW, kw, CHUNK)
    idxe3 = idx_e.reshape(NW, ke, CHUNK)
    src2 = sources.astype(jnp.int32).reshape(NW, B // NW)
    dst2 = destinations.astype(jnp.int32).reshape(NW, B // NW)

    g_w_rows, g_e_rows, dsg, dtg = _sc_gather(
        embW, node_emb, idxw3, idxe3, src2, dst2, delta_s, delta_t)

    o = 0
    sW = g_w_rows[o:o + B].reshape(B, 1, EMB); o += B
    tW = g_w_rows[o:o + B].reshape(B, 1, EMB); o += B
    n0w = g_w_rows[o:o + B * NEG].reshape(B, NEG, EMB); o += B * NEG
    n1w = g_w_rows[o:o + B * NEG].reshape(B, NEG, EMB); o += B * NEG
    hws = g_w_rows[o:o + B * HIST].reshape(B, HIST, EMB); o += B * HIST
    hwt = g_w_rows[o:o + B * HIST].reshape(B, HIST, EMB)
    o = 0
    p0 = g_e_rows[o:o + B].reshape(B, 1, EMB); o += B
    p1 = g_e_rows[o:o + B].reshape(B, 1, EMB); o += B
    n0 = g_e_rows[o:o + B * NEG].reshape(B, NEG, EMB); o += B * NEG
    n1 = g_e_rows[o:o + B * NEG].reshape(B, NEG, EMB)

    a2 = a.reshape(2, EMB)
    gwT = g_w.reshape(1, EMB)
    gb2 = g_b.reshape(1, 1)

    ew3, loss2 = _run_att(
        p0, p1, sW, tW, n0, n1, n0w, n1w, hws, hwt,
        e_t, sh_t, th_t, s_m, t_m,
        dsg.reshape(B, 1, 1), dtg.reshape(B, 1, 1), a2, gwT, gb2)

    tgsl_loss = loss2.reshape(())
    estimated_weights = ew3.reshape(B)
    return tgsl_loss, estimated_weights, node_emb


# speedup vs baseline: 1.4094x; 1.1165x over previous
import functools

import jax
import jax.numpy as jnp
from jax import lax
from jax.experimental import pallas as pl
from jax.experimental.pallas import tpu as pltpu
from jax.experimental.pallas import tpu_sc as plsc

NODE_DIM = 100000
FEAT_DIM = 128
EMB = 128
BATCH = 4096
HIST = 20
NEG = 5
INDEX = 200000
MAX_T = 100000.0
SIGMA = 1.0

NW = 32
CHUNK = 128
NBUF = 6

ROWS_W = BATCH * (4 + NEG * 2 + HIST * 2)
ROWS_E = BATCH * 2 + BATCH * NEG * 2


def _mlp_body(x_ref, w1_ref, b1_ref, w2_ref, b2_ref, w_ref, emb_ref, embw_ref):
    x = x_ref[...]
    h = jnp.maximum(
        jnp.dot(x, w1_ref[...], preferred_element_type=jnp.float32) + b1_ref[...],
        0.0)
    e = jnp.dot(h, w2_ref[...], preferred_element_type=jnp.float32) + b2_ref[...]
    emb_ref[...] = e
    embw_ref[...] = jnp.dot(e, w_ref[...], preferred_element_type=jnp.float32)


def _run_mlp(node_features, w1p, b1p, w2p, b2p, W):
    R = 1000
    G = NODE_DIM // R
    return pl.pallas_call(
        _mlp_body,
        grid=(G,),
        in_specs=[
            pl.BlockSpec((R, FEAT_DIM), lambda i: (i, 0)),
            pl.BlockSpec((FEAT_DIM, 128), lambda i: (0, 0)),
            pl.BlockSpec((1, 128), lambda i: (0, 0)),
            pl.BlockSpec((128, EMB), lambda i: (0, 0)),
            pl.BlockSpec((1, EMB), lambda i: (0, 0)),
            pl.BlockSpec((EMB, EMB), lambda i: (0, 0)),
        ],
        out_specs=[
            pl.BlockSpec((R, EMB), lambda i: (i, 0)),
            pl.BlockSpec((R, EMB), lambda i: (i, 0)),
        ],
        out_shape=[
            jax.ShapeDtypeStruct((NODE_DIM, EMB), jnp.float32),
            jax.ShapeDtypeStruct((NODE_DIM, EMB), jnp.float32),
        ],
    )(node_features, w1p, b1p, w2p, b2p, W)


def _sc_gather(embW, emb, idxw3, idxe3, src2, dst2, delta_s, delta_t):
    kw = idxw3.shape[1]
    ke = idxe3.shape[1]
    mesh = plsc.VectorSubcoreMesh(core_axis_name="c", subcore_axis_name="s")
    BN = BATCH * NEG
    BH = BATCH * HIST

    @functools.partial(
        pl.kernel,
        mesh=mesh,
        out_type=[
            jax.ShapeDtypeStruct((BATCH, EMB), jnp.float32),
            jax.ShapeDtypeStruct((BATCH, EMB), jnp.float32),
            jax.ShapeDtypeStruct((BN, EMB), jnp.float32),
            jax.ShapeDtypeStruct((BN, EMB), jnp.float32),
            jax.ShapeDtypeStruct((BH, EMB), jnp.float32),
            jax.ShapeDtypeStruct((BH, EMB), jnp.float32),
            jax.ShapeDtypeStruct((BATCH, EMB), jnp.float32),
            jax.ShapeDtypeStruct((BATCH, EMB), jnp.float32),
            jax.ShapeDtypeStruct((BN, EMB), jnp.float32),
            jax.ShapeDtypeStruct((BN, EMB), jnp.float32),
            jax.ShapeDtypeStruct((BATCH,), jnp.float32),
            jax.ShapeDtypeStruct((BATCH,), jnp.float32),
        ],
        scratch_types=[
            pltpu.VMEM((kw, CHUNK), jnp.int32),
            pltpu.VMEM((ke, CHUNK), jnp.int32),
            pltpu.VMEM((NBUF, CHUNK, EMB), jnp.float32),
            pltpu.VMEM((CHUNK,), jnp.int32),
            pltpu.VMEM((CHUNK,), jnp.float32),
            pltpu.SemaphoreType.DMA((NBUF,)),
            pltpu.SemaphoreType.DMA((NBUF,)),
            pltpu.SemaphoreType.DMA,
        ],
    )
    def k(embw_hbm, emb_hbm, idxw_hbm, idxe_hbm, src_hbm, dst_hbm,
          ds_hbm, dt_hbm,
          sw_out, tw_out, n0w_out, n1w_out, hws_out, hwt_out,
          p0_out, p1_out, n0_out, n1_out, dsg_out, dtg_out,
          idxw_v, idxe_v, bufs, didx, dval, gsems, osems, dsem):
        wid = lax.axis_index("s") * 2 + lax.axis_index("c")

        def seg_map(segs):
            def out_map(j, w):
                j0 = 0
                for n_j, ref in segs:
                    if j < j0 + n_j:
                        return ref, ((j - j0) * NW + w) * CHUNK
                    j0 += n_j
                raise AssertionError
            return out_map

        map_w = seg_map([(1, sw_out), (1, tw_out), (NEG, n0w_out),
                         (NEG, n1w_out), (HIST, hws_out), (HIST, hwt_out)])
        map_e = seg_map([(1, p0_out), (1, p1_out), (NEG, n0_out),
                         (NEG, n1_out)])

        LG = 3
        def gather_set(table_hbm, idx_v, n_chunks, out_map):
            handles_g = [None] * NBUF
            handles_o = [None] * NBUF
            for t in range(n_chunks + LG):
                u = t - LG
                if u >= 0:
                    slot = u % NBUF
                    handles_g[slot].wait()
                    ref, base = out_map(u, wid)
                    handles_o[slot] = pltpu.async_copy(
                        bufs.at[slot], ref.at[pl.ds(base, CHUNK)],
                        osems.at[slot])
                if t < n_chunks:
                    slot = t % NBUF
                    if handles_o[slot] is not None:
                        handles_o[slot].wait()
                        handles_o[slot] = None
                    handles_g[slot] = pltpu.async_copy(
                        table_hbm.at[idx_v.at[t]], bufs.at[slot],
                        gsems.at[slot])
            for slot in range(NBUF):
                if handles_o[slot] is not None:
                    handles_o[slot].wait()

        pltpu.sync_copy(idxw_hbm.at[wid], idxw_v)
        pltpu.sync_copy(idxe_hbm.at[wid], idxe_v)
        gather_set(embw_hbm, idxw_v, kw, map_w)
        gather_set(emb_hbm, idxe_v, ke, map_e)

        pltpu.sync_copy(src_hbm.at[wid], didx)
        pltpu.async_copy(ds_hbm.at[didx], dval, dsem).wait()
        pltpu.sync_copy(dval, dsg_out.at[pl.ds(wid * CHUNK, CHUNK)])
        pltpu.sync_copy(dst_hbm.at[wid], didx)
        pltpu.async_copy(dt_hbm.at[didx], dval, dsem).wait()
        pltpu.sync_copy(dval, dtg_out.at[pl.ds(wid * CHUNK, CHUNK)])

    return k(embW, emb, idxw3, idxe3, src2, dst2, delta_s, delta_t)


def _att_body(p0_ref, p1_ref, sw_ref, tw_ref, n0_ref, n1_ref, n0w_ref, n1w_ref,
              hws_ref, hwt_ref, et_ref, sht_ref, tht_ref, sm_ref, tm_ref,
              ds_ref, dt_ref, a2_ref, gw_ref, gb_ref, ew_ref, loss_ref):
    p0 = p0_ref[...]
    p1 = p1_ref[...]
    sw = sw_ref[...]
    tw = tw_ref[...]
    a_top = a2_ref[0:1, :].reshape(1, 1, EMB)
    a_bot = a2_ref[1:2, :].reshape(1, 1, EMB)
    gwv = gw_ref[...].reshape(1, 1, EMB)
    gbv = gb_ref[...].reshape(1, 1, 1)
    ds = ds_ref[...]
    dt_ = dt_ref[...]

    etn = et_ref[...] / MAX_T
    shn = sht_ref[...] / MAX_T
    thn = tht_ref[...] / MAX_T
    dts = jnp.abs(etn - shn)
    dtt = jnp.abs(etn - thn)

    hws = hws_ref[...]
    hwt = hwt_ref[...]
    swa = jnp.sum(sw * a_top, axis=2, keepdims=True)
    twa = jnp.sum(tw * a_top, axis=2, keepdims=True)
    hsa = jnp.sum(hws * a_bot, axis=2, keepdims=True)
    hta = jnp.sum(hwt * a_bot, axis=2, keepdims=True)

    def lrelu(x):
        return jnp.where(x >= 0, x, 0.2 * x)

    sims = lrelu(jnp.exp(-ds * dts) * (swa + hsa))
    simt = lrelu(jnp.exp(-ds * dtt) * (twa + hta))

    ms = jnp.max(sims, axis=1, keepdims=True)
    es = jnp.exp(sims - ms)
    atts = es / jnp.sum(es, axis=1, keepdims=True)
    mt = jnp.max(simt, axis=1, keepdims=True)
    et2 = jnp.exp(simt - mt)
    attt = et2 / jnp.sum(et2, axis=1, keepdims=True)

    shat_i = jnp.sum((atts * sm_ref[...]) * hws, axis=1, keepdims=True)
    that_i = jnp.sum((attt * tm_ref[...]) * hwt, axis=1, keepdims=True)

    mdts = jnp.mean(dts, axis=1, keepdims=True)
    mdtt = jnp.mean(dtt, axis=1, keepdims=True)
    gs = shat_i * jnp.exp(-ds * mdts)
    gt = that_i * jnp.exp(-dt_ * mdtt)
    gas = jnp.tanh(jnp.sum(gs * gwv, axis=2, keepdims=True) + gbv)
    gat = jnp.tanh(jnp.sum(gt * gwv, axis=2, keepdims=True) + gbv)
    mg = jnp.maximum(gas, gat)
    egs = jnp.exp(gas - mg)
    egt = jnp.exp(gat - mg)
    den = egs + egt
    shat = sw + (egs / den) * shat_i
    that = tw + (egt / den) * that_i

    pos = jnp.maximum(jnp.sum(p0 * p1, axis=2, keepdims=True), 0.0)
    negs = jnp.maximum(jnp.sum(p0 * n1_ref[...], axis=2, keepdims=True), 0.0)
    negt = jnp.maximum(jnp.sum(p1 * n0_ref[...], axis=2, keepdims=True), 0.0)

    psim = jnp.sum(shat * that, axis=2, keepdims=True)
    snsim = jnp.sum(shat * n1w_ref[...], axis=2, keepdims=True)
    tnsim = jnp.sum(that * n0w_ref[...], axis=2, keepdims=True)

    def sig(x):
        return 1.0 / (1.0 + jnp.exp(-x))

    pls = -jnp.log(sig(psim / SIGMA) + 1e-6) * (pos - 1.0) ** 2
    snl = -jnp.log(sig(-snsim / SIGMA) + 1e-6) * negs ** 2
    tnl = -jnp.log(sig(-tnsim / SIGMA) + 1e-6) * negt ** 2
    partial = (jnp.sum(pls) / INDEX
               + (jnp.sum(snl) + jnp.sum(tnl)) / (INDEX * NEG))

    i = pl.program_id(0)

    @pl.when(i == 0)
    def _():
        loss_ref[...] = jnp.zeros_like(loss_ref)

    loss_ref[...] += jnp.reshape(partial, (1, 1))
    ew_ref[...] = pos


def _run_att(p0, p1, sw, tw, n0, n1, n0w, n1w, hws, hwt,
             et, sht, tht, sm, tm, dsg, dtg, a2, gwT, gb2):
    TB = 128
    G = BATCH // TB
    b3 = lambda shp: pl.BlockSpec((TB,) + shp, lambda i: (i, 0, 0))
    full2 = lambda shp: pl.BlockSpec(shp, lambda i: (0, 0))
    return pl.pallas_call(
        _att_body,
        grid=(G,),
        in_specs=[
            b3((1, EMB)), b3((1, EMB)), b3((1, EMB)), b3((1, EMB)),
            b3((NEG, EMB)), b3((NEG, EMB)), b3((NEG, EMB)), b3((NEG, EMB)),
            b3((HIST, EMB)), b3((HIST, EMB)),
            b3((1, 1)), b3((HIST, 1)), b3((HIST, 1)),
            b3((HIST, 1)), b3((HIST, 1)),
            b3((1, 1)), b3((1, 1)),
            full2((2, EMB)), full2((1, EMB)), full2((1, 1)),
        ],
        out_specs=[
            b3((1, 1)),
            pl.BlockSpec((1, 1), lambda i: (0, 0)),
        ],
        out_shape=[
            jax.ShapeDtypeStruct((BATCH, 1, 1), jnp.float32),
            jax.ShapeDtypeStruct((1, 1), jnp.float32),
        ],
    )(p0, p1, sw, tw, n0, n1, n0w, n1w, hws, hwt,
      et, sht, tht, sm, tm, dsg, dtg, a2, gwT, gb2)


def kernel(sources, destinations, node_features, start_idx, end_idx,
           timestamps, s_h_nodes, t_h_nodes, s_h_times, t_h_times,
           s_his_mask, t_his_mask, neg_src_idx, neg_dst_idx,
           delta_s, delta_t, W, a, g_w, g_b,
           mlp_w1, mlp_b1, mlp_w2, mlp_b2):
    B = sources.shape[0]

    w1p = jnp.zeros((FEAT_DIM, 128), jnp.float32).at[:, :mlp_w1.shape[1]].set(mlp_w1)
    b1p = jnp.zeros((1, 128), jnp.float32).at[0, :mlp_b1.shape[0]].set(mlp_b1)
    w2p = jnp.zeros((128, EMB), jnp.float32).at[:mlp_w2.shape[0], :].set(mlp_w2)
    b2p = mlp_b2.reshape(1, EMB)

    node_emb, embW = _run_mlp(node_features, w1p, b1p, w2p, b2p, W)

    begin = end_idx - B
    sl = lambda x: lax.dynamic_slice_in_dim(x, begin, B, axis=0)
    e_t = sl(timestamps).reshape(B, 1, 1)
    sh_n = sl(s_h_nodes)
    th_n = sl(t_h_nodes)
    sh_t = sl(s_h_times).reshape(B, HIST, 1)
    th_t = sl(t_h_times).reshape(B, HIST, 1)
    s_m = sl(s_his_mask).reshape(B, HIST, 1)
    t_m = sl(t_his_mask).reshape(B, HIST, 1)

    idx_w = jnp.concatenate([
        sources, destinations, neg_src_idx, neg_dst_idx,
        sh_n.reshape(-1), th_n.reshape(-1),
    ]).astype(jnp.int32)
    idx_e = jnp.concatenate([
        sources, destinations, neg_src_idx, neg_dst_idx,
    ]).astype(jnp.int32)
    kw = idx_w.shape[0] // (NW * CHUNK)
    ke = idx_e.shape[0] // (NW * CHUNK)
    idxw3 = idx_w.reshape(kw, NW, CHUNK).transpose(1, 0, 2)
    idxe3 = idx_e.reshape(ke, NW, CHUNK).transpose(1, 0, 2)
    src2 = sources.astype(jnp.int32).reshape(NW, B // NW)
    dst2 = destinations.astype(jnp.int32).reshape(NW, B // NW)

    (sW, tW, n0w, n1w, hws, hwt, p0, p1, n0, n1, dsg, dtg) = _sc_gather(
        embW, node_emb, idxw3, idxe3, src2, dst2, delta_s, delta_t)
    sW = sW.reshape(B, 1, EMB)
    tW = tW.reshape(B, 1, EMB)
    n0w = n0w.reshape(B, NEG, EMB)
    n1w = n1w.reshape(B, NEG, EMB)
    hws = hws.reshape(B, HIST, EMB)
    hwt = hwt.reshape(B, HIST, EMB)
    p0 = p0.reshape(B, 1, EMB)
    p1 = p1.reshape(B, 1, EMB)
    n0 = n0.reshape(B, NEG, EMB)
    n1 = n1.reshape(B, NEG, EMB)

    a2 = a.reshape(2, EMB)
    gwT = g_w.reshape(1, EMB)
    gb2 = g_b.reshape(1, 1)

    ew3, loss2 = _run_att(
        p0, p1, sW, tW, n0, n1, n0w, n1w, hws, hwt,
        e_t, sh_t, th_t, s_m, t_m,
        dsg.reshape(B, 1, 1), dtg.reshape(B, 1, 1), a2, gwT, gb2)

    tgsl_loss = loss2.reshape(())
    estimated_weights = ew3.reshape(B)
    return tgsl_loss, estimated_weights, node_emb


# speedup vs baseline: 2.1288x; 1.5105x over previous
import functools

import jax
import jax.numpy as jnp
from jax import lax
from jax.experimental import pallas as pl
from jax.experimental.pallas import tpu as pltpu
from jax.experimental.pallas import tpu_sc as plsc

NODE_DIM = 100000
FEAT_DIM = 128
EMB = 128
BATCH = 4096
HIST = 20
NEG = 5
INDEX = 200000
MAX_T = 100000.0
SIGMA = 1.0

NW = 32
CHUNK = 128
NBUF = 6

ROWS_W = BATCH * (4 + NEG * 2 + HIST * 2)
ROWS_E = BATCH * 2 + BATCH * NEG * 2


def _mlp_body(x_ref, w1_ref, b1_ref, w2_ref, b2_ref, w_ref, emb_ref, embw_ref):
    x = x_ref[...]
    h = jnp.maximum(
        jnp.dot(x, w1_ref[...], preferred_element_type=jnp.float32) + b1_ref[...],
        0.0)
    e = jnp.dot(h, w2_ref[...], preferred_element_type=jnp.float32) + b2_ref[...]
    emb_ref[...] = e
    embw_ref[...] = jnp.dot(e, w_ref[...], preferred_element_type=jnp.float32)


def _run_mlp(node_features, w1p, b1p, w2p, b2p, W):
    R = 1000
    G = NODE_DIM // R
    return pl.pallas_call(
        _mlp_body,
        grid=(G,),
        in_specs=[
            pl.BlockSpec((R, FEAT_DIM), lambda i: (i, 0)),
            pl.BlockSpec((FEAT_DIM, 128), lambda i: (0, 0)),
            pl.BlockSpec((1, 128), lambda i: (0, 0)),
            pl.BlockSpec((128, EMB), lambda i: (0, 0)),
            pl.BlockSpec((1, EMB), lambda i: (0, 0)),
            pl.BlockSpec((EMB, EMB), lambda i: (0, 0)),
        ],
        out_specs=[
            pl.BlockSpec((R, EMB), lambda i: (i, 0)),
            pl.BlockSpec((R, EMB), lambda i: (i, 0)),
        ],
        out_shape=[
            jax.ShapeDtypeStruct((NODE_DIM, EMB), jnp.float32),
            jax.ShapeDtypeStruct((NODE_DIM, EMB), jnp.float32),
        ],
    )(node_features, w1p, b1p, w2p, b2p, W)


def _sc_gather(embW, emb, idxw3, idxe3, src2, dst2, delta_s, delta_t):
    kw = idxw3.shape[1]
    ke = idxe3.shape[1]
    mesh = plsc.VectorSubcoreMesh(core_axis_name="c", subcore_axis_name="s")
    BN = BATCH * NEG
    BH = BATCH * HIST

    @functools.partial(
        pl.kernel,
        mesh=mesh,
        out_type=[
            jax.ShapeDtypeStruct((BATCH, EMB), jnp.float32),
            jax.ShapeDtypeStruct((BATCH, EMB), jnp.float32),
            jax.ShapeDtypeStruct((BN, EMB), jnp.float32),
            jax.ShapeDtypeStruct((BN, EMB), jnp.float32),
            jax.ShapeDtypeStruct((BH, EMB), jnp.float32),
            jax.ShapeDtypeStruct((BH, EMB), jnp.float32),
            jax.ShapeDtypeStruct((BATCH, EMB), jnp.float32),
            jax.ShapeDtypeStruct((BATCH, EMB), jnp.float32),
            jax.ShapeDtypeStruct((BN, EMB), jnp.float32),
            jax.ShapeDtypeStruct((BN, EMB), jnp.float32),
            jax.ShapeDtypeStruct((BATCH,), jnp.float32),
            jax.ShapeDtypeStruct((BATCH,), jnp.float32),
        ],
        scratch_types=[
            pltpu.VMEM((kw, CHUNK), jnp.int32),
            pltpu.VMEM((ke, CHUNK), jnp.int32),
            pltpu.VMEM((NBUF, CHUNK, EMB), jnp.float32),
            pltpu.VMEM((CHUNK,), jnp.int32),
            pltpu.VMEM((CHUNK,), jnp.float32),
            pltpu.SemaphoreType.DMA((NBUF,)),
            pltpu.SemaphoreType.DMA((NBUF,)),
            pltpu.SemaphoreType.DMA,
        ],
    )
    def k(embw_hbm, emb_hbm, idxw_hbm, idxe_hbm, src_hbm, dst_hbm,
          ds_hbm, dt_hbm,
          sw_out, tw_out, n0w_out, n1w_out, hws_out, hwt_out,
          p0_out, p1_out, n0_out, n1_out, dsg_out, dtg_out,
          idxw_v, idxe_v, bufs, didx, dval, gsems, osems, dsem):
        wid = lax.axis_index("s") * 2 + lax.axis_index("c")

        def seg_map(segs):
            def out_map(j, w):
                j0 = 0
                for n_j, ref in segs:
                    if j < j0 + n_j:
                        return ref, ((j - j0) * NW + w) * CHUNK
                    j0 += n_j
                raise AssertionError
            return out_map

        map_w = seg_map([(1, sw_out), (1, tw_out), (NEG, n0w_out),
                         (NEG, n1w_out), (HIST, hws_out), (HIST, hwt_out)])
        map_e = seg_map([(1, p0_out), (1, p1_out), (NEG, n0_out),
                         (NEG, n1_out)])

        LG = 3
        def gather_set(table_hbm, idx_v, n_chunks, out_map):
            handles_g = [None] * NBUF
            handles_o = [None] * NBUF
            for t in range(n_chunks + LG):
                u = t - LG
                if u >= 0:
                    slot = u % NBUF
                    handles_g[slot].wait()
                    ref, base = out_map(u, wid)
                    handles_o[slot] = pltpu.async_copy(
                        bufs.at[slot], ref.at[pl.ds(base, CHUNK)],
                        osems.at[slot])
                if t < n_chunks:
                    slot = t % NBUF
                    if handles_o[slot] is not None:
                        handles_o[slot].wait()
                        handles_o[slot] = None
                    handles_g[slot] = pltpu.async_copy(
                        table_hbm.at[idx_v.at[t]], bufs.at[slot],
                        gsems.at[slot])
            for slot in range(NBUF):
                if handles_o[slot] is not None:
                    handles_o[slot].wait()

        pltpu.sync_copy(idxw_hbm.at[wid], idxw_v)
        pltpu.sync_copy(idxe_hbm.at[wid], idxe_v)
        gather_set(embw_hbm, idxw_v, kw, map_w)
        gather_set(emb_hbm, idxe_v, ke, map_e)

        pltpu.sync_copy(src_hbm.at[wid], didx)
        pltpu.async_copy(ds_hbm.at[didx], dval, dsem).wait()
        pltpu.sync_copy(dval, dsg_out.at[pl.ds(wid * CHUNK, CHUNK)])
        pltpu.sync_copy(dst_hbm.at[wid], didx)
        pltpu.async_copy(dt_hbm.at[didx], dval, dsem).wait()
        pltpu.sync_copy(dval, dtg_out.at[pl.ds(wid * CHUNK, CHUNK)])

    return k(embW, emb, idxw3, idxe3, src2, dst2, delta_s, delta_t)


def _att_body(p0_ref, p1_ref, sw_ref, tw_ref, n0_ref, n1_ref, n0w_ref, n1w_ref,
              hws_ref, hwt_ref, et_ref, sht_ref, tht_ref, sm_ref, tm_ref,
              ds_ref, dt_ref, a2_ref, gw_ref, gb_ref, ew_ref, loss_ref):
    p0 = p0_ref[...]
    p1 = p1_ref[...]
    sw = sw_ref[...]
    tw = tw_ref[...]
    a_top = a2_ref[0:1, :]
    a_bot = a2_ref[1:2, :]
    gwr = gw_ref[...]
    gb = gb_ref[...]
    ds = ds_ref[...]
    dt_ = dt_ref[...]
    hws = hws_ref[...]
    hwt = hwt_ref[...]

    etn = et_ref[...] / MAX_T
    dts = jnp.abs(etn - sht_ref[...] / MAX_T)
    dtt = jnp.abs(etn - tht_ref[...] / MAX_T)

    def seg(x, j):
        return x[:, j * EMB:(j + 1) * EMB]

    def rdot(x, row):
        return jnp.sum(x * row, axis=1, keepdims=True)

    swa = rdot(sw, a_top)
    twa = rdot(tw, a_top)
    hsa = jnp.concatenate([rdot(seg(hws, h), a_bot) for h in range(HIST)],
                          axis=1)
    hta = jnp.concatenate([rdot(seg(hwt, h), a_bot) for h in range(HIST)],
                          axis=1)

    def lrelu(x):
        return jnp.where(x >= 0, x, 0.2 * x)

    sims = lrelu(jnp.exp(-ds * dts) * (swa + hsa))
    simt = lrelu(jnp.exp(-ds * dtt) * (twa + hta))

    ms = jnp.max(sims, axis=1, keepdims=True)
    es = jnp.exp(sims - ms)
    atts = es / jnp.sum(es, axis=1, keepdims=True)
    mt = jnp.max(simt, axis=1, keepdims=True)
    et2 = jnp.exp(simt - mt)
    attt = et2 / jnp.sum(et2, axis=1, keepdims=True)

    ws = atts * sm_ref[...]
    wt = attt * tm_ref[...]
    shat_i = ws[:, 0:1] * seg(hws, 0)
    that_i = wt[:, 0:1] * seg(hwt, 0)
    for h in range(1, HIST):
        shat_i = shat_i + ws[:, h:h + 1] * seg(hws, h)
        that_i = that_i + wt[:, h:h + 1] * seg(hwt, h)

    mdts = jnp.mean(dts, axis=1, keepdims=True)
    mdtt = jnp.mean(dtt, axis=1, keepdims=True)
    gs = shat_i * jnp.exp(-ds * mdts)
    gt = that_i * jnp.exp(-dt_ * mdtt)
    gas = jnp.tanh(rdot(gs, gwr) + gb)
    gat = jnp.tanh(rdot(gt, gwr) + gb)
    mg = jnp.maximum(gas, gat)
    egs = jnp.exp(gas - mg)
    egt = jnp.exp(gat - mg)
    den = egs + egt
    shat = sw + (egs / den) * shat_i
    that = tw + (egt / den) * that_i

    pos = jnp.maximum(jnp.sum(p0 * p1, axis=1, keepdims=True), 0.0)
    n0 = n0_ref[...]
    n1 = n1_ref[...]
    n0w = n0w_ref[...]
    n1w = n1w_ref[...]
    dot_k = lambda x, wide: jnp.concatenate(
        [jnp.sum(x * seg(wide, k), axis=1, keepdims=True)
         for k in range(NEG)], axis=1)
    negs = jnp.maximum(dot_k(p0, n1), 0.0)
    negt = jnp.maximum(dot_k(p1, n0), 0.0)
    psim = jnp.sum(shat * that, axis=1, keepdims=True)
    snsim = dot_k(shat, n1w)
    tnsim = dot_k(that, n0w)

    def sig(x):
        return 1.0 / (1.0 + jnp.exp(-x))

    pls = -jnp.log(sig(psim / SIGMA) + 1e-6) * (pos - 1.0) ** 2
    snl = -jnp.log(sig(-snsim / SIGMA) + 1e-6) * negs ** 2
    tnl = -jnp.log(sig(-tnsim / SIGMA) + 1e-6) * negt ** 2
    partial = (jnp.sum(pls) / INDEX
               + (jnp.sum(snl) + jnp.sum(tnl)) / (INDEX * NEG))

    i = pl.program_id(0)

    @pl.when(i == 0)
    def _():
        loss_ref[...] = jnp.zeros_like(loss_ref)

    loss_ref[...] += jnp.reshape(partial, (1, 1))
    ew_ref[...] = pos


def _run_att(p0, p1, sw, tw, n0, n1, n0w, n1w, hws, hwt,
             et, sht, tht, sm, tm, dsg, dtg, a2, gwT, gb2):
    TB = 256
    G = BATCH // TB
    b2 = lambda w: pl.BlockSpec((TB, w), lambda i: (i, 0))
    full2 = lambda shp: pl.BlockSpec(shp, lambda i: (0, 0))
    return pl.pallas_call(
        _att_body,
        grid=(G,),
        in_specs=[
            b2(EMB), b2(EMB), b2(EMB), b2(EMB),
            b2(NEG * EMB), b2(NEG * EMB), b2(NEG * EMB), b2(NEG * EMB),
            b2(HIST * EMB), b2(HIST * EMB),
            b2(1), b2(HIST), b2(HIST), b2(HIST), b2(HIST),
            b2(1), b2(1),
            full2((2, EMB)), full2((1, EMB)), full2((1, 1)),
        ],
        out_specs=[
            b2(1),
            pl.BlockSpec((1, 1), lambda i: (0, 0)),
        ],
        out_shape=[
            jax.ShapeDtypeStruct((BATCH, 1), jnp.float32),
            jax.ShapeDtypeStruct((1, 1), jnp.float32),
        ],
    )(p0, p1, sw, tw, n0, n1, n0w, n1w, hws, hwt,
      et, sht, tht, sm, tm, dsg, dtg, a2, gwT, gb2)


def kernel(sources, destinations, node_features, start_idx, end_idx,
           timestamps, s_h_nodes, t_h_nodes, s_h_times, t_h_times,
           s_his_mask, t_his_mask, neg_src_idx, neg_dst_idx,
           delta_s, delta_t, W, a, g_w, g_b,
           mlp_w1, mlp_b1, mlp_w2, mlp_b2):
    B = sources.shape[0]

    w1p = jnp.zeros((FEAT_DIM, 128), jnp.float32).at[:, :mlp_w1.shape[1]].set(mlp_w1)
    b1p = jnp.zeros((1, 128), jnp.float32).at[0, :mlp_b1.shape[0]].set(mlp_b1)
    w2p = jnp.zeros((128, EMB), jnp.float32).at[:mlp_w2.shape[0], :].set(mlp_w2)
    b2p = mlp_b2.reshape(1, EMB)

    node_emb, embW = _run_mlp(node_features, w1p, b1p, w2p, b2p, W)

    begin = end_idx - B
    sl = lambda x: lax.dynamic_slice_in_dim(x, begin, B, axis=0)
    e_t = sl(timestamps).reshape(B, 1)
    sh_n = sl(s_h_nodes)
    th_n = sl(t_h_nodes)
    sh_t = sl(s_h_times)
    th_t = sl(t_h_times)
    s_m = sl(s_his_mask)
    t_m = sl(t_his_mask)

    idx_w = jnp.concatenate([
        sources, destinations, neg_src_idx, neg_dst_idx,
        sh_n.reshape(-1), th_n.reshape(-1),
    ]).astype(jnp.int32)
    idx_e = jnp.concatenate([
        sources, destinations, neg_src_idx, neg_dst_idx,
    ]).astype(jnp.int32)
    kw = idx_w.shape[0] // (NW * CHUNK)
    ke = idx_e.shape[0] // (NW * CHUNK)
    idxw3 = idx_w.reshape(kw, NW, CHUNK).transpose(1, 0, 2)
    idxe3 = idx_e.reshape(ke, NW, CHUNK).transpose(1, 0, 2)
    src2 = sources.astype(jnp.int32).reshape(NW, B // NW)
    dst2 = destinations.astype(jnp.int32).reshape(NW, B // NW)

    (sW, tW, n0w, n1w, hws, hwt, p0, p1, n0, n1, dsg, dtg) = _sc_gather(
        embW, node_emb, idxw3, idxe3, src2, dst2, delta_s, delta_t)
    n0w = n0w.reshape(B, NEG * EMB)
    n1w = n1w.reshape(B, NEG * EMB)
    hws = hws.reshape(B, HIST * EMB)
    hwt = hwt.reshape(B, HIST * EMB)
    n0 = n0.reshape(B, NEG * EMB)
    n1 = n1.reshape(B, NEG * EMB)

    a2 = a.reshape(2, EMB)
    gwT = g_w.reshape(1, EMB)
    gb2 = g_b.reshape(1, 1)

    ew2, loss2 = _run_att(
        p0, p1, sW, tW, n0, n1, n0w, n1w, hws, hwt,
        e_t, sh_t, th_t, s_m, t_m,
        dsg.reshape(B, 1), dtg.reshape(B, 1), a2, gwT, gb2)

    tgsl_loss = loss2.reshape(())
    estimated_weights = ew2.reshape(B)
    return tgsl_loss, estimated_weights, node_emb


# speedup vs baseline: 2.9440x; 1.3830x over previous
import functools

import jax
import jax.numpy as jnp
from jax import lax
from jax.experimental import pallas as pl
from jax.experimental.pallas import tpu as pltpu
from jax.experimental.pallas import tpu_sc as plsc

NODE_DIM = 100000
FEAT_DIM = 128
EMB = 128
BATCH = 4096
HIST = 20
NEG = 5
INDEX = 200000
MAX_T = 100000.0
SIGMA = 1.0

NW = 32
CHUNK = 128
NBUF = 6


def _mlp_body(x_ref, w1_ref, b1_ref, w2_ref, b2_ref, w_ref, emb_ref, embw_ref):
    x = x_ref[...]
    h = jnp.maximum(
        jnp.dot(x, w1_ref[...], preferred_element_type=jnp.float32) + b1_ref[...],
        0.0)
    e = jnp.dot(h, w2_ref[...], preferred_element_type=jnp.float32) + b2_ref[...]
    emb_ref[...] = e
    embw_ref[...] = jnp.dot(e, w_ref[...], preferred_element_type=jnp.float32)


def _run_mlp(node_features, w1p, b1p, w2p, b2p, W):
    R = 1000
    G = NODE_DIM // R
    return pl.pallas_call(
        _mlp_body,
        grid=(G,),
        in_specs=[
            pl.BlockSpec((R, FEAT_DIM), lambda i: (i, 0)),
            pl.BlockSpec((FEAT_DIM, 128), lambda i: (0, 0)),
            pl.BlockSpec((1, 128), lambda i: (0, 0)),
            pl.BlockSpec((128, EMB), lambda i: (0, 0)),
            pl.BlockSpec((1, EMB), lambda i: (0, 0)),
            pl.BlockSpec((EMB, EMB), lambda i: (0, 0)),
        ],
        out_specs=[
            pl.BlockSpec((R, EMB), lambda i: (i, 0)),
            pl.BlockSpec((R, EMB), lambda i: (i, 0)),
        ],
        out_shape=[
            jax.ShapeDtypeStruct((NODE_DIM, EMB), jnp.float32),
            jax.ShapeDtypeStruct((NODE_DIM, EMB), jnp.float32),
        ],
    )(node_features, w1p, b1p, w2p, b2p, W)


def _sc_gather(embW, emb, srcB, dstB, negsT, negdT, shT, thT,
               delta_s, delta_t):
    kw = 2 + 2 * NEG + 2 * HIST
    ke = 2 + 2 * NEG
    mesh = plsc.VectorSubcoreMesh(core_axis_name="c", subcore_axis_name="s")
    BN = BATCH * NEG
    BH = BATCH * HIST

    @functools.partial(
        pl.kernel,
        mesh=mesh,
        out_type=[
            jax.ShapeDtypeStruct((BATCH, EMB), jnp.float32),
            jax.ShapeDtypeStruct((BATCH, EMB), jnp.float32),
            jax.ShapeDtypeStruct((BN, EMB), jnp.float32),
            jax.ShapeDtypeStruct((BN, EMB), jnp.float32),
            jax.ShapeDtypeStruct((BH, EMB), jnp.float32),
            jax.ShapeDtypeStruct((BH, EMB), jnp.float32),
            jax.ShapeDtypeStruct((BATCH, EMB), jnp.float32),
            jax.ShapeDtypeStruct((BATCH, EMB), jnp.float32),
            jax.ShapeDtypeStruct((BN, EMB), jnp.float32),
            jax.ShapeDtypeStruct((BN, EMB), jnp.float32),
            jax.ShapeDtypeStruct((BATCH,), jnp.float32),
            jax.ShapeDtypeStruct((BATCH,), jnp.float32),
        ],
        scratch_types=[
            pltpu.VMEM((kw, CHUNK), jnp.int32),
            pltpu.VMEM((NBUF, CHUNK, EMB), jnp.float32),
            pltpu.VMEM((CHUNK,), jnp.float32),
            pltpu.SemaphoreType.DMA((NBUF,)),
            pltpu.SemaphoreType.DMA((NBUF,)),
            pltpu.SemaphoreType.DMA,
        ],
    )
    def k(embw_hbm, emb_hbm, src_hbm, dst_hbm, negs_hbm, negd_hbm,
          shr_hbm, thr_hbm, ds_hbm, dt_hbm,
          sw_out, tw_out, n0w_out, n1w_out, hws_out, hwt_out,
          p0_out, p1_out, n0_out, n1_out, dsg_out, dtg_out,
          idxw_v, bufs, dval, gsems, osems, dsem):
        wid = lax.axis_index("s") * 2 + lax.axis_index("c")

        def seg_map(segs):
            def out_map(j, w):
                j0 = 0
                for n_j, ref in segs:
                    if j < j0 + n_j:
                        return ref, (w * n_j + (j - j0)) * CHUNK
                    j0 += n_j
                raise AssertionError
            return out_map

        map_w = seg_map([(1, sw_out), (1, tw_out), (NEG, n0w_out),
                         (NEG, n1w_out), (HIST, hws_out), (HIST, hwt_out)])
        map_e = seg_map([(1, p0_out), (1, p1_out), (NEG, n0_out),
                         (NEG, n1_out)])

        LG = 3
        def gather_set(table_hbm, idx_v, n_chunks, out_map, bufs, nbuf):
            handles_g = [None] * nbuf
            handles_o = [None] * nbuf
            for t in range(n_chunks + LG):
                u = t - LG
                if u >= 0:
                    slot = u % nbuf
                    handles_g[slot].wait()
                    ref, base = out_map(u, wid)
                    handles_o[slot] = pltpu.async_copy(
                        bufs.at[slot], ref.at[pl.ds(base, CHUNK)],
                        osems.at[slot])
                if t < n_chunks:
                    slot = t % nbuf
                    if handles_o[slot] is not None:
                        handles_o[slot].wait()
                        handles_o[slot] = None
                    handles_g[slot] = pltpu.async_copy(
                        table_hbm.at[idx_v.at[t]], bufs.at[slot],
                        gsems.at[slot])
            for slot in range(nbuf):
                if handles_o[slot] is not None:
                    handles_o[slot].wait()

        pltpu.sync_copy(src_hbm.at[wid], idxw_v.at[0])
        pltpu.sync_copy(dst_hbm.at[wid], idxw_v.at[1])
        pltpu.sync_copy(negs_hbm.at[wid], idxw_v.at[pl.ds(2, NEG)])
        pltpu.sync_copy(negd_hbm.at[wid], idxw_v.at[pl.ds(2 + NEG, NEG)])
        pltpu.sync_copy(shr_hbm.at[wid], idxw_v.at[pl.ds(12, HIST)])
        pltpu.sync_copy(thr_hbm.at[wid], idxw_v.at[pl.ds(12 + HIST, HIST)])
        gather_set(embw_hbm, idxw_v, kw, map_w, bufs, NBUF)
        gather_set(emb_hbm, idxw_v, ke, map_e, bufs, NBUF)

        pltpu.async_copy(ds_hbm.at[idxw_v.at[0]], dval, dsem).wait()
        pltpu.sync_copy(dval, dsg_out.at[pl.ds(wid * CHUNK, CHUNK)])
        pltpu.async_copy(dt_hbm.at[idxw_v.at[1]], dval, dsem).wait()
        pltpu.sync_copy(dval, dtg_out.at[pl.ds(wid * CHUNK, CHUNK)])

    return k(embW, emb, srcB, dstB, negsT, negdT, shT, thT,
             delta_s, delta_t)


def _att_body(p0_ref, p1_ref, sw_ref, tw_ref, n0_ref, n1_ref, n0w_ref, n1w_ref,
              hws_ref, hwt_ref, et_ref, sht_ref, tht_ref, sm_ref, tm_ref,
              ds_ref, dt_ref, a2_ref, gw_ref, gb_ref, ew_ref, loss_ref):
    f32 = jnp.float32
    p0 = p0_ref[...]
    p1 = p1_ref[...]
    sw = sw_ref[...]
    tw = tw_ref[...]
    a_top = a2_ref[0:1, :]
    a_bot = a2_ref[1:2, :]
    gwr = gw_ref[...]
    gb = gb_ref[...]
    ds = ds_ref[...]
    dt_ = dt_ref[...]
    hws = [hws_ref[h] for h in range(HIST)]
    hwt = [hwt_ref[h] for h in range(HIST)]

    etn = et_ref[...] / MAX_T
    dts = jnp.abs(etn - sht_ref[...] / MAX_T)
    dtt = jnp.abs(etn - tht_ref[...] / MAX_T)

    def rdot(x, row):
        return jnp.sum(x * row, axis=1, keepdims=True)

    swa = rdot(sw, a_top)
    twa = rdot(tw, a_top)
    hsa = jnp.concatenate([rdot(hws[h], a_bot) for h in range(HIST)],
                          axis=1)
    hta = jnp.concatenate([rdot(hwt[h], a_bot) for h in range(HIST)],
                          axis=1)

    def lrelu(x):
        return jnp.where(x >= 0, x, 0.2 * x)

    sims = lrelu(jnp.exp(-ds * dts) * (swa + hsa))
    simt = lrelu(jnp.exp(-ds * dtt) * (twa + hta))

    ms = jnp.max(sims, axis=1, keepdims=True)
    es = jnp.exp(sims - ms)
    atts = es / jnp.sum(es, axis=1, keepdims=True)
    mt = jnp.max(simt, axis=1, keepdims=True)
    et2 = jnp.exp(simt - mt)
    attt = et2 / jnp.sum(et2, axis=1, keepdims=True)

    ws = atts * sm_ref[...]
    wt = attt * tm_ref[...]
    shat_i = ws[:, 0:1] * hws[0]
    that_i = wt[:, 0:1] * hwt[0]
    for h in range(1, HIST):
        shat_i = shat_i + ws[:, h:h + 1] * hws[h]
        that_i = that_i + wt[:, h:h + 1] * hwt[h]

    mdts = jnp.mean(dts, axis=1, keepdims=True)
    mdtt = jnp.mean(dtt, axis=1, keepdims=True)
    gs = shat_i * jnp.exp(-ds * mdts)
    gt = that_i * jnp.exp(-dt_ * mdtt)
    gas = jnp.tanh(rdot(gs, gwr) + gb)
    gat = jnp.tanh(rdot(gt, gwr) + gb)
    mg = jnp.maximum(gas, gat)
    egs = jnp.exp(gas - mg)
    egt = jnp.exp(gat - mg)
    den = egs + egt
    shat = sw + (egs / den) * shat_i
    that = tw + (egt / den) * that_i

    pos = jnp.maximum(jnp.sum(p0 * p1, axis=1, keepdims=True), 0.0)
    dot_k = lambda x, ref3: jnp.concatenate(
        [jnp.sum(x * ref3[k], axis=1, keepdims=True)
         for k in range(NEG)], axis=1)
    negs = jnp.maximum(dot_k(p0, n1_ref), 0.0)
    negt = jnp.maximum(dot_k(p1, n0_ref), 0.0)
    psim = jnp.sum(shat * that, axis=1, keepdims=True)
    snsim = dot_k(shat, n1w_ref)
    tnsim = dot_k(that, n0w_ref)

    def sig(x):
        return 1.0 / (1.0 + jnp.exp(-x))

    pls = -jnp.log(sig(psim / SIGMA) + 1e-6) * (pos - 1.0) ** 2
    snl = -jnp.log(sig(-snsim / SIGMA) + 1e-6) * negs ** 2
    tnl = -jnp.log(sig(-tnsim / SIGMA) + 1e-6) * negt ** 2
    partial = (jnp.sum(pls) / INDEX
               + (jnp.sum(snl) + jnp.sum(tnl)) / (INDEX * NEG))

    i = pl.program_id(0)

    @pl.when(i == 0)
    def _():
        loss_ref[...] = jnp.zeros_like(loss_ref)

    loss_ref[...] += jnp.reshape(partial, (1, 1))
    ew_ref[...] = pos


def _run_att(p0, p1, sw, tw, n0, n1, n0w, n1w, hws, hwt,
             et, sht, tht, sm, tm, dsg, dtg, a2, gwT, gb2):
    TB = 256
    G = BATCH // TB
    b2 = lambda w: pl.BlockSpec((TB, w), lambda i: (i, 0))
    b3 = lambda lead: pl.BlockSpec((lead, TB, EMB), lambda i: (0, i, 0))
    full2 = lambda shp: pl.BlockSpec(shp, lambda i: (0, 0))
    return pl.pallas_call(
        _att_body,
        grid=(G,),
        in_specs=[
            b2(EMB), b2(EMB), b2(EMB), b2(EMB),
            b3(NEG), b3(NEG), b3(NEG), b3(NEG),
            b3(HIST), b3(HIST),
            b2(1), b2(HIST), b2(HIST), b2(HIST), b2(HIST),
            b2(1), b2(1),
            full2((2, EMB)), full2((1, EMB)), full2((1, 1)),
        ],
        out_specs=[
            b2(1),
            pl.BlockSpec((1, 1), lambda i: (0, 0)),
        ],
        out_shape=[
            jax.ShapeDtypeStruct((BATCH, 1), jnp.float32),
            jax.ShapeDtypeStruct((1, 1), jnp.float32),
        ],
    )(p0, p1, sw, tw, n0, n1, n0w, n1w, hws, hwt,
      et, sht, tht, sm, tm, dsg, dtg, a2, gwT, gb2)


def kernel(sources, destinations, node_features, start_idx, end_idx,
           timestamps, s_h_nodes, t_h_nodes, s_h_times, t_h_times,
           s_his_mask, t_his_mask, neg_src_idx, neg_dst_idx,
           delta_s, delta_t, W, a, g_w, g_b,
           mlp_w1, mlp_b1, mlp_w2, mlp_b2):
    B = sources.shape[0]

    w1p = jnp.zeros((FEAT_DIM, 128), jnp.float32).at[:, :mlp_w1.shape[1]].set(mlp_w1)
    b1p = jnp.zeros((1, 128), jnp.float32).at[0, :mlp_b1.shape[0]].set(mlp_b1)
    w2p = jnp.zeros((128, EMB), jnp.float32).at[:mlp_w2.shape[0], :].set(mlp_w2)
    b2p = mlp_b2.reshape(1, EMB)

    node_emb, embW = _run_mlp(node_features, w1p, b1p, w2p, b2p, W)

    begin = end_idx - B
    sl = lambda x: lax.dynamic_slice_in_dim(x, begin, B, axis=0)
    e_t = sl(timestamps).reshape(B, 1)
    sh_n = sl(s_h_nodes)
    th_n = sl(t_h_nodes)
    sh_t = sl(s_h_times)
    th_t = sl(t_h_times)
    s_m = sl(s_his_mask)
    t_m = sl(t_his_mask)

    srcB = sources.astype(jnp.int32).reshape(NW, CHUNK)
    dstB = destinations.astype(jnp.int32).reshape(NW, CHUNK)
    negsT = neg_src_idx.astype(jnp.int32).reshape(B, NEG).T.reshape(NW, NEG, CHUNK)
    negdT = neg_dst_idx.astype(jnp.int32).reshape(B, NEG).T.reshape(NW, NEG, CHUNK)
    shT = sh_n.astype(jnp.int32).T.reshape(NW, HIST, CHUNK)
    thT = th_n.astype(jnp.int32).T.reshape(NW, HIST, CHUNK)

    (sW, tW, n0w, n1w, hws, hwt, p0, p1, n0, n1, dsg, dtg) = _sc_gather(
        embW, node_emb, srcB, dstB, negsT, negdT, shT, thT,
        delta_s, delta_t)
    n0w = n0w.reshape(NEG, B, EMB)
    n1w = n1w.reshape(NEG, B, EMB)
    hws = hws.reshape(HIST, B, EMB)
    hwt = hwt.reshape(HIST, B, EMB)
    n0 = n0.reshape(NEG, B, EMB)
    n1 = n1.reshape(NEG, B, EMB)

    a2 = a.reshape(2, EMB)
    gwT = g_w.reshape(1, EMB)
    gb2 = g_b.reshape(1, 1)

    ew2, loss2 = _run_att(
        p0, p1, sW, tW, n0, n1, n0w, n1w, hws, hwt,
        e_t, sh_t, th_t, s_m, t_m,
        dsg.reshape(B, 1), dtg.reshape(B, 1), a2, gwT, gb2)

    tgsl_loss = loss2.reshape(())
    estimated_weights = ew2.reshape(B)
    return tgsl_loss, estimated_weights, node_emb


# speedup vs baseline: 3.3581x; 1.1406x over previous
import functools

import jax
import jax.numpy as jnp
from jax import lax
from jax.experimental import pallas as pl
from jax.experimental.pallas import tpu as pltpu
from jax.experimental.pallas import tpu_sc as plsc

NODE_DIM = 100000
FEAT_DIM = 128
EMB = 128
BATCH = 4096
HIST = 20
NEG = 5
INDEX = 200000
MAX_T = 100000.0
SIGMA = 1.0

NW = 32
CHUNK = 128
NBUF = 6


def _mlp_body(x_ref, w1_ref, b1_ref, w2_ref, b2_ref, w_ref, emb_ref, embw_ref):
    x = x_ref[...]
    h = jnp.maximum(
        jnp.dot(x, w1_ref[...], preferred_element_type=jnp.float32) + b1_ref[...],
        0.0)
    e = jnp.dot(h, w2_ref[...], preferred_element_type=jnp.float32) + b2_ref[...]
    emb_ref[...] = e
    embw_ref[...] = jnp.dot(e, w_ref[...], preferred_element_type=jnp.float32)


def _run_mlp(node_features, w1p, b1p, w2p, b2p, W):
    R = 2000
    G = NODE_DIM // R
    return pl.pallas_call(
        _mlp_body,
        grid=(G,),
        in_specs=[
            pl.BlockSpec((R, FEAT_DIM), lambda i: (i, 0)),
            pl.BlockSpec((FEAT_DIM, 128), lambda i: (0, 0)),
            pl.BlockSpec((1, 128), lambda i: (0, 0)),
            pl.BlockSpec((128, EMB), lambda i: (0, 0)),
            pl.BlockSpec((1, EMB), lambda i: (0, 0)),
            pl.BlockSpec((EMB, EMB), lambda i: (0, 0)),
        ],
        out_specs=[
            pl.BlockSpec((R, EMB), lambda i: (i, 0)),
            pl.BlockSpec((R, EMB), lambda i: (i, 0)),
        ],
        out_shape=[
            jax.ShapeDtypeStruct((NODE_DIM, EMB), jnp.float32),
            jax.ShapeDtypeStruct((NODE_DIM, EMB), jnp.float32),
        ],
    )(node_features, w1p, b1p, w2p, b2p, W)


def _sc_gather(embW, emb, srcB, dstB, negsT, negdT, shT, thT,
               delta_s, delta_t):
    kw = 2 + 2 * NEG + 2 * HIST
    ke = 2 + 2 * NEG
    mesh = plsc.VectorSubcoreMesh(core_axis_name="c", subcore_axis_name="s")
    BN = BATCH * NEG
    BH = BATCH * HIST

    @functools.partial(
        pl.kernel,
        mesh=mesh,
        out_type=[
            jax.ShapeDtypeStruct((BATCH, EMB), jnp.float32),
            jax.ShapeDtypeStruct((BATCH, EMB), jnp.float32),
            jax.ShapeDtypeStruct((BN, EMB), jnp.float32),
            jax.ShapeDtypeStruct((BN, EMB), jnp.float32),
            jax.ShapeDtypeStruct((BH, EMB), jnp.float32),
            jax.ShapeDtypeStruct((BH, EMB), jnp.float32),
            jax.ShapeDtypeStruct((BATCH, EMB), jnp.float32),
            jax.ShapeDtypeStruct((BATCH, EMB), jnp.float32),
            jax.ShapeDtypeStruct((BN, EMB), jnp.float32),
            jax.ShapeDtypeStruct((BN, EMB), jnp.float32),
            jax.ShapeDtypeStruct((BATCH,), jnp.float32),
            jax.ShapeDtypeStruct((BATCH,), jnp.float32),
        ],
        scratch_types=[
            pltpu.VMEM((kw, CHUNK), jnp.int32),
            pltpu.VMEM((NBUF, CHUNK, EMB), jnp.float32),
            pltpu.VMEM((CHUNK,), jnp.float32),
            pltpu.SemaphoreType.DMA((NBUF,)),
            pltpu.SemaphoreType.DMA((NBUF,)),
            pltpu.SemaphoreType.DMA,
        ],
    )
    def k(embw_hbm, emb_hbm, src_hbm, dst_hbm, negs_hbm, negd_hbm,
          shr_hbm, thr_hbm, ds_hbm, dt_hbm,
          sw_out, tw_out, n0w_out, n1w_out, hws_out, hwt_out,
          p0_out, p1_out, n0_out, n1_out, dsg_out, dtg_out,
          idxw_v, bufs, dval, gsems, osems, dsem):
        wid = lax.axis_index("s") * 2 + lax.axis_index("c")

        def seg_map(segs):
            def out_map(j, w):
                j0 = 0
                for n_j, ref in segs:
                    if j < j0 + n_j:
                        return ref, (w * n_j + (j - j0)) * CHUNK
                    j0 += n_j
                raise AssertionError
            return out_map

        map_w = seg_map([(1, sw_out), (1, tw_out), (NEG, n0w_out),
                         (NEG, n1w_out), (HIST, hws_out), (HIST, hwt_out)])
        map_e = seg_map([(1, p0_out), (1, p1_out), (NEG, n0_out),
                         (NEG, n1_out)])

        LG = 3
        def gather_set(table_hbm, idx_v, n_chunks, out_map, bufs, nbuf):
            handles_g = [None] * nbuf
            handles_o = [None] * nbuf
            for t in range(n_chunks + LG):
                u = t - LG
                if u >= 0:
                    slot = u % nbuf
                    handles_g[slot].wait()
                    ref, base = out_map(u, wid)
                    handles_o[slot] = pltpu.async_copy(
                        bufs.at[slot], ref.at[pl.ds(base, CHUNK)],
                        osems.at[slot])
                if t < n_chunks:
                    slot = t % nbuf
                    if handles_o[slot] is not None:
                        handles_o[slot].wait()
                        handles_o[slot] = None
                    handles_g[slot] = pltpu.async_copy(
                        table_hbm.at[idx_v.at[t]], bufs.at[slot],
                        gsems.at[slot])
            for slot in range(nbuf):
                if handles_o[slot] is not None:
                    handles_o[slot].wait()

        pltpu.sync_copy(src_hbm.at[wid], idxw_v.at[0])
        pltpu.sync_copy(dst_hbm.at[wid], idxw_v.at[1])
        pltpu.sync_copy(negs_hbm.at[wid], idxw_v.at[pl.ds(2, NEG)])
        pltpu.sync_copy(negd_hbm.at[wid], idxw_v.at[pl.ds(2 + NEG, NEG)])
        pltpu.sync_copy(shr_hbm.at[wid], idxw_v.at[pl.ds(12, HIST)])
        pltpu.sync_copy(thr_hbm.at[wid], idxw_v.at[pl.ds(12 + HIST, HIST)])
        gather_set(embw_hbm, idxw_v, kw, map_w, bufs, NBUF)
        gather_set(emb_hbm, idxw_v, ke, map_e, bufs, NBUF)

        pltpu.async_copy(ds_hbm.at[idxw_v.at[0]], dval, dsem).wait()
        pltpu.sync_copy(dval, dsg_out.at[pl.ds(wid * CHUNK, CHUNK)])
        pltpu.async_copy(dt_hbm.at[idxw_v.at[1]], dval, dsem).wait()
        pltpu.sync_copy(dval, dtg_out.at[pl.ds(wid * CHUNK, CHUNK)])

    return k(embW, emb, srcB, dstB, negsT, negdT, shT, thT,
             delta_s, delta_t)


def _att_body(p0_ref, p1_ref, sw_ref, tw_ref, n0_ref, n1_ref, n0w_ref, n1w_ref,
              hws_ref, hwt_ref, et_ref, sht_ref, tht_ref, sm_ref, tm_ref,
              ds_ref, dt_ref, a2_ref, gw_ref, gb_ref, ew_ref, loss_ref):
    f32 = jnp.float32
    p0 = p0_ref[...]
    p1 = p1_ref[...]
    sw = sw_ref[...]
    tw = tw_ref[...]
    a_top = a2_ref[0:1, :]
    a_bot = a2_ref[1:2, :]
    gwr = gw_ref[...]
    gb = gb_ref[...]
    ds = ds_ref[...]
    dt_ = dt_ref[...]
    hws = [hws_ref[h] for h in range(HIST)]
    hwt = [hwt_ref[h] for h in range(HIST)]

    etn = et_ref[...] / MAX_T
    dts = jnp.abs(etn - sht_ref[...] / MAX_T)
    dtt = jnp.abs(etn - tht_ref[...] / MAX_T)

    def rdot(x, row):
        return jnp.sum(x * row, axis=1, keepdims=True)

    swa = rdot(sw, a_top)
    twa = rdot(tw, a_top)
    hsa = jnp.concatenate([rdot(hws[h], a_bot) for h in range(HIST)],
                          axis=1)
    hta = jnp.concatenate([rdot(hwt[h], a_bot) for h in range(HIST)],
                          axis=1)

    def lrelu(x):
        return jnp.where(x >= 0, x, 0.2 * x)

    sims = lrelu(jnp.exp(-ds * dts) * (swa + hsa))
    simt = lrelu(jnp.exp(-ds * dtt) * (twa + hta))

    ms = jnp.max(sims, axis=1, keepdims=True)
    es = jnp.exp(sims - ms)
    atts = es / jnp.sum(es, axis=1, keepdims=True)
    mt = jnp.max(simt, axis=1, keepdims=True)
    et2 = jnp.exp(simt - mt)
    attt = et2 / jnp.sum(et2, axis=1, keepdims=True)

    ws = atts * sm_ref[...]
    wt = attt * tm_ref[...]
    shat_i = ws[:, 0:1] * hws[0]
    that_i = wt[:, 0:1] * hwt[0]
    for h in range(1, HIST):
        shat_i = shat_i + ws[:, h:h + 1] * hws[h]
        that_i = that_i + wt[:, h:h + 1] * hwt[h]

    mdts = jnp.mean(dts, axis=1, keepdims=True)
    mdtt = jnp.mean(dtt, axis=1, keepdims=True)
    gs = shat_i * jnp.exp(-ds * mdts)
    gt = that_i * jnp.exp(-dt_ * mdtt)
    gas = jnp.tanh(rdot(gs, gwr) + gb)
    gat = jnp.tanh(rdot(gt, gwr) + gb)
    mg = jnp.maximum(gas, gat)
    egs = jnp.exp(gas - mg)
    egt = jnp.exp(gat - mg)
    den = egs + egt
    shat = sw + (egs / den) * shat_i
    that = tw + (egt / den) * that_i

    pos = jnp.maximum(jnp.sum(p0 * p1, axis=1, keepdims=True), 0.0)
    dot_k = lambda x, ref3: jnp.concatenate(
        [jnp.sum(x * ref3[k], axis=1, keepdims=True)
         for k in range(NEG)], axis=1)
    negs = jnp.maximum(dot_k(p0, n1_ref), 0.0)
    negt = jnp.maximum(dot_k(p1, n0_ref), 0.0)
    psim = jnp.sum(shat * that, axis=1, keepdims=True)
    snsim = dot_k(shat, n1w_ref)
    tnsim = dot_k(that, n0w_ref)

    def sig(x):
        return 1.0 / (1.0 + jnp.exp(-x))

    pls = -jnp.log(sig(psim / SIGMA) + 1e-6) * (pos - 1.0) ** 2
    snl = -jnp.log(sig(-snsim / SIGMA) + 1e-6) * negs ** 2
    tnl = -jnp.log(sig(-tnsim / SIGMA) + 1e-6) * negt ** 2
    partial = (jnp.sum(pls) / INDEX
               + (jnp.sum(snl) + jnp.sum(tnl)) / (INDEX * NEG))

    i = pl.program_id(0)

    @pl.when(i == 0)
    def _():
        loss_ref[...] = jnp.zeros_like(loss_ref)

    loss_ref[...] += jnp.reshape(partial, (1, 1))
    ew_ref[...] = pos


def _run_att(p0, p1, sw, tw, n0, n1, n0w, n1w, hws, hwt,
             et, sht, tht, sm, tm, dsg, dtg, a2, gwT, gb2):
    TB = 512
    G = BATCH // TB
    b2 = lambda w: pl.BlockSpec((TB, w), lambda i: (i, 0))
    b3 = lambda lead: pl.BlockSpec((lead, TB, EMB), lambda i: (0, i, 0))
    full2 = lambda shp: pl.BlockSpec(shp, lambda i: (0, 0))
    return pl.pallas_call(
        _att_body,
        grid=(G,),
        in_specs=[
            b2(EMB), b2(EMB), b2(EMB), b2(EMB),
            b3(NEG), b3(NEG), b3(NEG), b3(NEG),
            b3(HIST), b3(HIST),
            b2(1), b2(HIST), b2(HIST), b2(HIST), b2(HIST),
            b2(1), b2(1),
            full2((2, EMB)), full2((1, EMB)), full2((1, 1)),
        ],
        out_specs=[
            b2(1),
            pl.BlockSpec((1, 1), lambda i: (0, 0)),
        ],
        out_shape=[
            jax.ShapeDtypeStruct((BATCH, 1), jnp.float32),
            jax.ShapeDtypeStruct((1, 1), jnp.float32),
        ],
    )(p0, p1, sw, tw, n0, n1, n0w, n1w, hws, hwt,
      et, sht, tht, sm, tm, dsg, dtg, a2, gwT, gb2)


def kernel(sources, destinations, node_features, start_idx, end_idx,
           timestamps, s_h_nodes, t_h_nodes, s_h_times, t_h_times,
           s_his_mask, t_his_mask, neg_src_idx, neg_dst_idx,
           delta_s, delta_t, W, a, g_w, g_b,
           mlp_w1, mlp_b1, mlp_w2, mlp_b2):
    B = sources.shape[0]

    w1p = jnp.zeros((FEAT_DIM, 128), jnp.float32).at[:, :mlp_w1.shape[1]].set(mlp_w1)
    b1p = jnp.zeros((1, 128), jnp.float32).at[0, :mlp_b1.shape[0]].set(mlp_b1)
    w2p = jnp.zeros((128, EMB), jnp.float32).at[:mlp_w2.shape[0], :].set(mlp_w2)
    b2p = mlp_b2.reshape(1, EMB)

    node_emb, embW = _run_mlp(node_features, w1p, b1p, w2p, b2p, W)

    begin = end_idx - B
    sl = lambda x: lax.dynamic_slice_in_dim(x, begin, B, axis=0)
    e_t = sl(timestamps).reshape(B, 1)
    sh_n = sl(s_h_nodes)
    th_n = sl(t_h_nodes)
    sh_t = sl(s_h_times)
    th_t = sl(t_h_times)
    s_m = sl(s_his_mask)
    t_m = sl(t_his_mask)

    srcB = sources.astype(jnp.int32).reshape(NW, CHUNK)
    dstB = destinations.astype(jnp.int32).reshape(NW, CHUNK)
    negsT = neg_src_idx.astype(jnp.int32).reshape(B, NEG).T.reshape(NW, NEG, CHUNK)
    negdT = neg_dst_idx.astype(jnp.int32).reshape(B, NEG).T.reshape(NW, NEG, CHUNK)
    shT = sh_n.astype(jnp.int32).T.reshape(NW, HIST, CHUNK)
    thT = th_n.astype(jnp.int32).T.reshape(NW, HIST, CHUNK)

    (sW, tW, n0w, n1w, hws, hwt, p0, p1, n0, n1, dsg, dtg) = _sc_gather(
        embW, node_emb, srcB, dstB, negsT, negdT, shT, thT,
        delta_s, delta_t)
    n0w = n0w.reshape(NEG, B, EMB)
    n1w = n1w.reshape(NEG, B, EMB)
    hws = hws.reshape(HIST, B, EMB)
    hwt = hwt.reshape(HIST, B, EMB)
    n0 = n0.reshape(NEG, B, EMB)
    n1 = n1.reshape(NEG, B, EMB)

    a2 = a.reshape(2, EMB)
    gwT = g_w.reshape(1, EMB)
    gb2 = g_b.reshape(1, 1)

    ew2, loss2 = _run_att(
        p0, p1, sW, tW, n0, n1, n0w, n1w, hws, hwt,
        e_t, sh_t, th_t, s_m, t_m,
        dsg.reshape(B, 1), dtg.reshape(B, 1), a2, gwT, gb2)

    tgsl_loss = loss2.reshape(())
    estimated_weights = ew2.reshape(B)
    return tgsl_loss, estimated_weights, node_emb
